# baseline (speedup 1.0000x reference)
import functools

import jax
import jax.numpy as jnp
from jax import lax
from jax.experimental import pallas as pl
from jax.experimental.pallas import tpu as pltpu
from jax.experimental.pallas import tpu_sc as plsc

_NUM_CORES = 2
_NUM_SUBCORES = 16
_NUM_WORKERS = _NUM_CORES * _NUM_SUBCORES


def kernel(input_ids, word_embeddings):
    B, S = input_ids.shape
    V, H = word_embeddings.shape
    s_per_w = S // _NUM_WORKERS

    mesh = plsc.VectorSubcoreMesh(core_axis_name="c", subcore_axis_name="s")

    @functools.partial(
        pl.kernel,
        mesh=mesh,
        out_type=jax.ShapeDtypeStruct((S, B * H), jnp.float32),
        scratch_types=[
            pltpu.VMEM((s_per_w,), jnp.int32),
            pltpu.VMEM((s_per_w, H), jnp.float32),
            pltpu.SemaphoreType.DMA,
        ],
    )
    def _emb(ids_hbm, tab_hbm, out_hbm, idx_v, rows_v, sem):
        wid = lax.axis_index("s") * _NUM_CORES + lax.axis_index("c")
        base_s = wid * s_per_w
        for b in range(B):
            pltpu.sync_copy(ids_hbm.at[b, pl.ds(base_s, s_per_w)], idx_v)
            pltpu.async_copy(tab_hbm.at[idx_v], rows_v, sem).wait()
            pltpu.sync_copy(
                rows_v, out_hbm.at[pl.ds(base_s, s_per_w), pl.ds(b * H, H)]
            )

    out = _emb(input_ids, word_embeddings)
    return out.reshape(S, B, H)


# speedup vs baseline: 1.0100x; 1.0100x over previous
import functools

import jax
import jax.numpy as jnp
from jax import lax
from jax.experimental import pallas as pl
from jax.experimental.pallas import tpu as pltpu
from jax.experimental.pallas import tpu_sc as plsc

_NUM_CORES = 2
_NUM_SUBCORES = 16
_NUM_WORKERS = _NUM_CORES * _NUM_SUBCORES
_K = 32
_NBUF = 3


def kernel(input_ids, word_embeddings):
    B, S = input_ids.shape
    V, H = word_embeddings.shape
    s_per_w = S // _NUM_WORKERS
    n_j = s_per_w // _K
    T = B * n_j

    mesh = plsc.VectorSubcoreMesh(core_axis_name="c", subcore_axis_name="s")

    scratch = [pltpu.VMEM((B, s_per_w), jnp.int32)]
    scratch += [pltpu.VMEM((_K, H), jnp.float32) for _ in range(_NBUF)]
    scratch += [pltpu.SemaphoreType.DMA for _ in range(2 * _NBUF)]

    @functools.partial(
        pl.kernel,
        mesh=mesh,
        out_type=jax.ShapeDtypeStruct((S, B * H), jnp.float32),
        scratch_types=scratch,
    )
    def _emb(ids_hbm, tab_hbm, out_hbm, idx_slab, *rest):
        bufs = rest[:_NBUF]
        gsems = rest[_NBUF:2 * _NBUF]
        osems = rest[2 * _NBUF:]
        wid = lax.axis_index("s") * _NUM_CORES + lax.axis_index("c")
        base_s = wid * s_per_w
        for b in range(B):
            pltpu.sync_copy(ids_hbm.at[b, pl.ds(base_s, s_per_w)], idx_slab.at[b])

        def chunk(t):
            b, j = divmod(t, n_j)
            idx = idx_slab.at[b, pl.ds(j * _K, _K)]
            out = out_hbm.at[pl.ds(base_s + j * _K, _K), pl.ds(b * H, H)]
            return idx, out

        hg = [None] * _NBUF
        ho = [None] * _NBUF
        for t in range(min(_NBUF, T)):
            idx, _ = chunk(t)
            hg[t] = pltpu.async_copy(tab_hbm.at[idx], bufs[t], gsems[t])
        for t in range(T):
            i = t % _NBUF
            hg[i].wait()
            _, outsl = chunk(t)
            ho[i] = pltpu.async_copy(bufs[i], outsl, osems[i])
            nxt = t + _NBUF
            if nxt < T:
                ho[i].wait()
                idx, _ = chunk(nxt)
                hg[i] = pltpu.async_copy(tab_hbm.at[idx], bufs[i], gsems[i])
        for t in range(max(0, T - _NBUF), T):
            ho[t % _NBUF].wait()

    out = _emb(input_ids, word_embeddings)
    return out.reshape(S, B, H)


# speedup vs baseline: 1.9174x; 1.8985x over previous
import functools

import jax
import jax.numpy as jnp
from jax import lax
from jax.experimental import pallas as pl
from jax.experimental.pallas import tpu as pltpu
from jax.experimental.pallas import tpu_sc as plsc

_NUM_CORES = 2
_NUM_SUBCORES = 16
_NUM_WORKERS = _NUM_CORES * _NUM_SUBCORES
_K = 32
_NBUF = 3


def kernel(input_ids, word_embeddings):
    B, S = input_ids.shape
    V, H = word_embeddings.shape
    s_per_w = S // _NUM_WORKERS
    n_j = s_per_w // _K
    T = B * n_j

    mesh = plsc.VectorSubcoreMesh(core_axis_name="c", subcore_axis_name="s")

    scratch = [pltpu.VMEM((B, s_per_w), jnp.int32)]
    scratch += [pltpu.VMEM((_K, H), jnp.float32) for _ in range(_NBUF)]
    scratch += [pltpu.SemaphoreType.DMA for _ in range(2 * _NBUF)]

    @functools.partial(
        pl.kernel,
        mesh=mesh,
        out_type=jax.ShapeDtypeStruct((S, B, H), jnp.float32),
        scratch_types=scratch,
    )
    def _emb(ids_hbm, tab_hbm, out_hbm, idx_slab, *rest):
        bufs = rest[:_NBUF]
        gsems = rest[_NBUF:2 * _NBUF]
        osems = rest[2 * _NBUF:]
        wid = lax.axis_index("s") * _NUM_CORES + lax.axis_index("c")
        base_s = wid * s_per_w
        for b in range(B):
            pltpu.sync_copy(ids_hbm.at[b, pl.ds(base_s, s_per_w)], idx_slab.at[b])

        def chunk(t):
            b, j = divmod(t, n_j)
            idx = idx_slab.at[b, pl.ds(j * _K, _K)]
            out = out_hbm.at[pl.ds(base_s + j * _K, _K), b, :]
            return idx, out

        hg = [None] * _NBUF
        ho = [None] * _NBUF
        for t in range(min(_NBUF, T)):
            idx, _ = chunk(t)
            hg[t] = pltpu.async_copy(tab_hbm.at[idx], bufs[t], gsems[t])
        for t in range(T):
            i = t % _NBUF
            hg[i].wait()
            _, outsl = chunk(t)
            ho[i] = pltpu.async_copy(bufs[i], outsl, osems[i])
            nxt = t + _NBUF
            if nxt < T:
                ho[i].wait()
                idx, _ = chunk(nxt)
                hg[i] = pltpu.async_copy(tab_hbm.at[idx], bufs[i], gsems[i])
        for t in range(max(0, T - _NBUF), T):
            ho[t % _NBUF].wait()

    return _emb(input_ids, word_embeddings)


# speedup vs baseline: 1.9606x; 1.0225x over previous
import functools

import jax
import jax.numpy as jnp
from jax import lax
from jax.experimental import pallas as pl
from jax.experimental.pallas import tpu as pltpu
from jax.experimental.pallas import tpu_sc as plsc

_NUM_CORES = 2
_NUM_SUBCORES = 16
_NUM_WORKERS = _NUM_CORES * _NUM_SUBCORES
_K = 32
_NBUF = 3


def kernel(input_ids, word_embeddings):
    B, S = input_ids.shape
    V, H = word_embeddings.shape
    s_per_w = S // _NUM_WORKERS
    n_j = s_per_w // _K
    T = B * n_j

    mesh = plsc.VectorSubcoreMesh(core_axis_name="c", subcore_axis_name="s")

    scratch = [pltpu.VMEM((B, s_per_w), jnp.int32)]
    scratch += [pltpu.VMEM((_K, H), jnp.float32) for _ in range(_NBUF)]
    scratch += [pltpu.SemaphoreType.DMA for _ in range(2 * _NBUF + 1)]

    @functools.partial(
        pl.kernel,
        mesh=mesh,
        out_type=jax.ShapeDtypeStruct((S, B, H), jnp.float32),
        scratch_types=scratch,
    )
    def _emb(ids_hbm, tab_hbm, out_hbm, idx_slab, *rest):
        bufs = rest[:_NBUF]
        gsems = rest[_NBUF:2 * _NBUF]
        osems = rest[2 * _NBUF:3 * _NBUF]
        isem = rest[3 * _NBUF]
        wid = lax.axis_index("s") * _NUM_CORES + lax.axis_index("c")
        base_s = wid * s_per_w
        ih = [
            pltpu.async_copy(
                ids_hbm.at[b, pl.ds(base_s, s_per_w)], idx_slab.at[b], isem
            )
            for b in range(B)
        ]
        for h in ih:
            h.wait()

        def chunk(t):
            b, j = divmod(t, n_j)
            idx = idx_slab.at[b, pl.ds(j * _K, _K)]
            out = out_hbm.at[pl.ds(base_s + j * _K, _K), b, :]
            return idx, out

        def fire_gather(t):
            idx, _ = chunk(t)
            i = t % _NBUF
            return pltpu.async_copy(tab_hbm.at[idx], bufs[i], gsems[i])

        hg = [None] * _NBUF
        ho = [None] * _NBUF
        for t in range(min(_NBUF - 1, T)):
            hg[t % _NBUF] = fire_gather(t)
        for t in range(T):
            i = t % _NBUF
            nxt = t + _NBUF - 1
            if nxt < T:
                j = nxt % _NBUF
                if ho[j] is not None:
                    ho[j].wait()
                    ho[j] = None
                hg[j] = fire_gather(nxt)
            hg[i].wait()
            _, outsl = chunk(t)
            ho[i] = pltpu.async_copy(bufs[i], outsl, osems[i])
        for h in ho:
            if h is not None:
                h.wait()

    return _emb(input_ids, word_embeddings)


# speedup vs baseline: 1.9611x; 1.0002x over previous
import functools

import jax
import jax.numpy as jnp
from jax import lax
from jax.experimental import pallas as pl
from jax.experimental.pallas import tpu as pltpu
from jax.experimental.pallas import tpu_sc as plsc

_NUM_CORES = 2
_NUM_SUBCORES = 16
_NUM_WORKERS = _NUM_CORES * _NUM_SUBCORES
_K = 16
_NBUF = 6
_LEAD = 3


def kernel(input_ids, word_embeddings):
    B, S = input_ids.shape
    V, H = word_embeddings.shape
    s_per_w = S // _NUM_WORKERS
    n_j = s_per_w // _K
    T = B * n_j

    mesh = plsc.VectorSubcoreMesh(core_axis_name="c", subcore_axis_name="s")

    scratch = [pltpu.VMEM((B, s_per_w), jnp.int32)]
    scratch += [pltpu.VMEM((_K, H), jnp.float32) for _ in range(_NBUF)]
    scratch += [pltpu.SemaphoreType.DMA for _ in range(2 * _NBUF + 1)]

    @functools.partial(
        pl.kernel,
        mesh=mesh,
        out_type=jax.ShapeDtypeStruct((S, B, H), jnp.float32),
        scratch_types=scratch,
    )
    def _emb(ids_hbm, tab_hbm, out_hbm, idx_slab, *rest):
        bufs = rest[:_NBUF]
        gsems = rest[_NBUF:2 * _NBUF]
        osems = rest[2 * _NBUF:3 * _NBUF]
        isem = rest[3 * _NBUF]
        wid = lax.axis_index("s") * _NUM_CORES + lax.axis_index("c")
        base_s = wid * s_per_w
        ih = [
            pltpu.async_copy(
                ids_hbm.at[b, pl.ds(base_s, s_per_w)], idx_slab.at[b], isem
            )
            for b in range(B)
        ]
        for h in ih:
            h.wait()

        def chunk(t):
            b, j = divmod(t, n_j)
            idx = idx_slab.at[b, pl.ds(j * _K, _K)]
            out = out_hbm.at[pl.ds(base_s + j * _K, _K), b, :]
            return idx, out

        def fire_gather(t):
            idx, _ = chunk(t)
            i = t % _NBUF
            return pltpu.async_copy(tab_hbm.at[idx], bufs[i], gsems[i])

        hg = [None] * _NBUF
        ho = [None] * _NBUF
        for t in range(min(_LEAD, T)):
            hg[t % _NBUF] = fire_gather(t)
        for t in range(T):
            i = t % _NBUF
            nxt = t + _LEAD
            if nxt < T:
                j = nxt % _NBUF
                if ho[j] is not None:
                    ho[j].wait()
                    ho[j] = None
                hg[j] = fire_gather(nxt)
            hg[i].wait()
            _, outsl = chunk(t)
            ho[i] = pltpu.async_copy(bufs[i], outsl, osems[i])
        for h in ho:
            if h is not None:
                h.wait()

    return _emb(input_ids, word_embeddings)


# speedup vs baseline: 1.9692x; 1.0041x over previous
import functools

import jax
import jax.numpy as jnp
from jax import lax
from jax.experimental import pallas as pl
from jax.experimental.pallas import tpu as pltpu
from jax.experimental.pallas import tpu_sc as plsc

_NUM_CORES = 2
_NUM_SUBCORES = 16
_NUM_WORKERS = _NUM_CORES * _NUM_SUBCORES
_K = 32
_NBUF = 3


def kernel(input_ids, word_embeddings):
    B, S = input_ids.shape
    V, H = word_embeddings.shape
    s_per_w = S // _NUM_WORKERS
    n_j = s_per_w // _K
    T = B * n_j

    mesh = plsc.VectorSubcoreMesh(core_axis_name="c", subcore_axis_name="s")

    scratch = [pltpu.VMEM((B, s_per_w), jnp.int32)]
    scratch += [pltpu.VMEM((_K, H), jnp.float32) for _ in range(_NBUF)]
    scratch += [pltpu.SemaphoreType.DMA for _ in range(2 * _NBUF + 1)]

    @functools.partial(
        pl.kernel,
        mesh=mesh,
        out_type=jax.ShapeDtypeStruct((S, B, H), jnp.float32),
        scratch_types=scratch,
    )
    def _emb(ids_hbm, tab_hbm, out_hbm, idx_slab, *rest):
        bufs = rest[:_NBUF]
        gsems = rest[_NBUF:2 * _NBUF]
        osems = rest[2 * _NBUF:3 * _NBUF]
        isem = rest[3 * _NBUF]
        wid = lax.axis_index("s") * _NUM_CORES + lax.axis_index("c")
        base_s = wid * s_per_w
        ih = [
            pltpu.async_copy(
                ids_hbm.at[b, pl.ds(base_s, s_per_w)], idx_slab.at[b], isem
            )
            for b in range(B)
        ]
        ids_ready = [False] * B

        def need_ids(b):
            if not ids_ready[b]:
                ih[b].wait()
                ids_ready[b] = True

        def chunk(t):
            b, j = divmod(t, n_j)
            idx = idx_slab.at[b, pl.ds(j * _K, _K)]
            out = out_hbm.at[pl.ds(base_s + j * _K, _K), b, :]
            return idx, out

        def fire_gather(t):
            need_ids(t // n_j)
            idx, _ = chunk(t)
            i = t % _NBUF
            return pltpu.async_copy(tab_hbm.at[idx], bufs[i], gsems[i])

        hg = [None] * _NBUF
        ho = [None] * _NBUF
        for t in range(min(_NBUF - 1, T)):
            hg[t % _NBUF] = fire_gather(t)
        for t in range(T):
            i = t % _NBUF
            nxt = t + _NBUF - 1
            if nxt < T:
                j = nxt % _NBUF
                if ho[j] is not None:
                    ho[j].wait()
                    ho[j] = None
                hg[j] = fire_gather(nxt)
            hg[i].wait()
            _, outsl = chunk(t)
            ho[i] = pltpu.async_copy(bufs[i], outsl, osems[i])
        for h in ho:
            if h is not None:
                h.wait()

    return _emb(input_ids, word_embeddings)
